# baseline (speedup 1.0000x reference)
import jax
import jax.numpy as jnp
from jax import lax
from jax.experimental import pallas as pl
from jax.experimental.pallas import tpu as pltpu
from jax.experimental.pallas import tpu_sc as plsc

RCR = 5.2
NEG_ETA = -16.0
NUM_SPECIES = 4
P = 16
L = 16
B, A = 16, 256
ROWS = B * A
NC, NS = 2, 16
NW = NC * NS
RPW = ROWS // NW
SUB = 64
NSUB = RPW // SUB
CHUNKS = A // L
JAM = 4
OUTW = NUM_SPECIES * P
SLAB = SUB * A

_FC4 = (0.9999795, -0.091211826, 0.0027640709,
        -3.2557324e-05, 1.6667427e-07)


def _sc_body(d_hbm, s_hbm, out_hbm, dr0, dr1, dbuf, fbuf, sbuf, srow16,
             obuf, sem0, sem1):
    cid = lax.axis_index("c")
    sid = lax.axis_index("s")
    w = sid * NC + cid
    row0 = w * RPW
    bt = row0 // A

    cp0 = pltpu.async_copy(d_hbm.at[pl.ds(row0 * A, SLAB)], dr0, sem0)
    cp1 = pltpu.async_copy(d_hbm.at[pl.ds((row0 + SUB) * A, SLAB)], dr1,
                           sem1)
    pltpu.sync_copy(s_hbm.at[pl.ds(bt * A, A)], srow16)

    iota = lax.iota(jnp.int32, L)
    zf = jnp.zeros((L,), jnp.float32)
    zi = jnp.zeros((L,), jnp.int32)

    perms = [(iota + k) & (P - 1) for k in range(P)]
    shrots = [0.9 + 0.26875 * pm.astype(jnp.float32) for pm in perms]

    for c in range(CHUNKS):
        sv = srow16[pl.ds(c * L, L)]
        srow16[pl.ds(c * L, L)] = (sv - 1) * P

    def zero_body(i, zc):
        ivec = zi + i * (8 * L)
        for k in range(8):
            plsc.store_scatter(obuf, [ivec + (iota + k * L)], zf)
        return zc

    lax.fori_loop(0, RPW * OUTW // (8 * L), zero_body, 0)

    for sub, drows, cp in ((0, dr0, cp0), (1, dr1, cp1)):
        cp.wait()

        def row_body(r, cnt):
            roff = pl.multiple_of(r * A, A)
            obase = (sub * SUB + r) * OUTW

            for j in range(0, CHUNKS, JAM):
                ks = range(JAM)
                dvs = [drows[pl.ds(roff + (j + k) * L, L)] for k in ks]
                valids = [(dv > 0.0) & (dv < RCR) for dv in dvs]
                pops = [plsc.all_reduce_population_count(v)
                        for v in valids]
                us = [dv * dv for dv in dvs]
                accs = [u * _FC4[4] + _FC4[3] for u in us]
                for coef in (_FC4[2], _FC4[1], _FC4[0]):
                    accs = [a * u for a, u in zip(accs, us)]
                    accs = [a + coef for a in accs]
                svs = [srow16[pl.ds((j + k) * L, L)] + obase for k in ks]
                css = [plsc.cumsum(v.astype(jnp.int32)) for v in valids]
                for k in ks:
                    pos = cnt + css[k] - 1
                    plsc.store_scatter(dbuf, [pos], dvs[k],
                                       mask=valids[k])
                    plsc.store_scatter(fbuf, [pos], accs[k],
                                       mask=valids[k])
                    plsc.store_scatter(sbuf, [pos], svs[k],
                                       mask=valids[k])
                    cnt = cnt + pops[k]
            return cnt

        cnt = lax.fori_loop(0, SUB, row_body, zi)

        for pq in range(2):
            pad = (cnt + pq * L) + iota
            plsc.store_scatter(dbuf, [pad], zf)
            plsc.store_scatter(fbuf, [pad], zf)
            plsc.store_scatter(sbuf, [pad], zi)

        n = cnt[0]
        niter = (n + (2 * L - 1)) // (2 * L)

        def group_body(g, gc):
            base = pl.multiple_of(g * (2 * L), 2 * L)
            for h in (0, L):
                dv16 = dbuf[pl.ds(base + h, L)]
                fv16 = fbuf[pl.ds(base + h, L)]
                sv16 = sbuf[pl.ds(base + h, L)]
                for k in range(P):
                    t = dv16 - shrots[k]
                    gv = jnp.exp((t * t) * NEG_ETA) * fv16
                    plsc.addupdate_scatter(obuf, [sv16 + perms[k]], gv)
            return gc

        lax.fori_loop(0, niter, group_body, 0)

    pltpu.sync_copy(obuf, out_hbm.at[pl.ds(row0 * OUTW, RPW * OUTW)])


def _make_sc_call():
    mesh = plsc.VectorSubcoreMesh(
        core_axis_name="c", subcore_axis_name="s", num_cores=NC, num_subcores=NS
    )
    return pl.kernel(
        _sc_body,
        out_type=jax.ShapeDtypeStruct((ROWS * OUTW,), jnp.float32),
        mesh=mesh,
        compiler_params=pltpu.CompilerParams(needs_layout_passes=False),
        scratch_types=[
            pltpu.VMEM((SLAB,), jnp.float32),
            pltpu.VMEM((SLAB,), jnp.float32),
            pltpu.VMEM((SLAB + 2 * L,), jnp.float32),
            pltpu.VMEM((SLAB + 2 * L,), jnp.float32),
            pltpu.VMEM((SLAB + 2 * L,), jnp.int32),
            pltpu.VMEM((A,), jnp.int32),
            pltpu.VMEM((RPW * OUTW,), jnp.float32),
            pltpu.SemaphoreType.DMA,
            pltpu.SemaphoreType.DMA,
        ],
    )


def kernel(distance_matrices_batch, num_species_batch):
    d = distance_matrices_batch.reshape(ROWS * A)
    s = num_species_batch.astype(jnp.int32).reshape(B * A)
    out = _make_sc_call()(d, s)
    return out.reshape(B, A, OUTW)


# speedup vs baseline: 1.0265x; 1.0265x over previous
import jax
import jax.numpy as jnp
from jax import lax
from jax.experimental import pallas as pl
from jax.experimental.pallas import tpu as pltpu
from jax.experimental.pallas import tpu_sc as plsc

RCR = 5.2
NEG_ETA = -16.0
NUM_SPECIES = 4
P = 16
L = 16
B, A = 16, 256
ROWS = B * A
NC, NS = 2, 16
NW = NC * NS
RPW = ROWS // NW
SUB = 64
NSUB = RPW // SUB
CHUNKS = A // L
JAM = 8
OUTW = NUM_SPECIES * P
SLAB = SUB * A

_FC4 = (0.9999795, -0.091211826, 0.0027640709,
        -3.2557324e-05, 1.6667427e-07)


def _sc_body(d_hbm, s_hbm, out_hbm, dr0, dr1, dbuf, fbuf, sbuf, srow16,
             obuf, sem0, sem1):
    cid = lax.axis_index("c")
    sid = lax.axis_index("s")
    w = sid * NC + cid
    row0 = w * RPW
    bt = row0 // A

    cp0 = pltpu.async_copy(d_hbm.at[pl.ds(row0 * A, SLAB)], dr0, sem0)
    cp1 = pltpu.async_copy(d_hbm.at[pl.ds((row0 + SUB) * A, SLAB)], dr1,
                           sem1)
    pltpu.sync_copy(s_hbm.at[pl.ds(bt * A, A)], srow16)

    iota = lax.iota(jnp.int32, L)
    zf = jnp.zeros((L,), jnp.float32)
    zi = jnp.zeros((L,), jnp.int32)

    perms = [(iota + k) & (P - 1) for k in range(P)]
    shrots = [0.9 + 0.26875 * pm.astype(jnp.float32) for pm in perms]

    for c in range(CHUNKS):
        sv = srow16[pl.ds(c * L, L)]
        srow16[pl.ds(c * L, L)] = (sv - 1) * P

    def zero_body(i, zc):
        ivec = zi + i * (8 * L)
        for k in range(8):
            plsc.store_scatter(obuf, [ivec + (iota + k * L)], zf)
        return zc

    lax.fori_loop(0, RPW * OUTW // (8 * L), zero_body, 0)

    for sub, drows, cp in ((0, dr0, cp0), (1, dr1, cp1)):
        cp.wait()

        def row_body(r, cnt):
            roff = pl.multiple_of(r * A, A)
            obase = (sub * SUB + r) * OUTW

            for j in range(0, CHUNKS, JAM):
                ks = range(JAM)
                dvs = [drows[pl.ds(roff + (j + k) * L, L)] for k in ks]
                valids = [(dv > 0.0) & (dv < RCR) for dv in dvs]
                pops = [plsc.all_reduce_population_count(v)
                        for v in valids]
                us = [dv * dv for dv in dvs]
                accs = [u * _FC4[4] + _FC4[3] for u in us]
                for coef in (_FC4[2], _FC4[1], _FC4[0]):
                    accs = [a * u for a, u in zip(accs, us)]
                    accs = [a + coef for a in accs]
                svs = [srow16[pl.ds((j + k) * L, L)] + obase for k in ks]
                css = [plsc.cumsum(v.astype(jnp.int32)) for v in valids]
                for k in ks:
                    pos = cnt + css[k] - 1
                    plsc.store_scatter(dbuf, [pos], dvs[k],
                                       mask=valids[k])
                    plsc.store_scatter(fbuf, [pos], accs[k],
                                       mask=valids[k])
                    plsc.store_scatter(sbuf, [pos], svs[k],
                                       mask=valids[k])
                    cnt = cnt + pops[k]
            return cnt

        cnt = lax.fori_loop(0, SUB, row_body, zi)

        for pq in range(2):
            pad = (cnt + pq * L) + iota
            plsc.store_scatter(dbuf, [pad], zf)
            plsc.store_scatter(fbuf, [pad], zf)
            plsc.store_scatter(sbuf, [pad], zi)

        n = cnt[0]
        niter = (n + (2 * L - 1)) // (2 * L)

        def group_body(g, gc):
            base = pl.multiple_of(g * (2 * L), 2 * L)
            for h in (0, L):
                dv16 = dbuf[pl.ds(base + h, L)]
                fv16 = fbuf[pl.ds(base + h, L)]
                sv16 = sbuf[pl.ds(base + h, L)]
                for k in range(P):
                    t = dv16 - shrots[k]
                    gv = jnp.exp((t * t) * NEG_ETA) * fv16
                    plsc.addupdate_scatter(obuf, [sv16 + perms[k]], gv)
            return gc

        lax.fori_loop(0, niter, group_body, 0)

    pltpu.sync_copy(obuf, out_hbm.at[pl.ds(row0 * OUTW, RPW * OUTW)])


def _make_sc_call():
    mesh = plsc.VectorSubcoreMesh(
        core_axis_name="c", subcore_axis_name="s", num_cores=NC, num_subcores=NS
    )
    return pl.kernel(
        _sc_body,
        out_type=jax.ShapeDtypeStruct((ROWS * OUTW,), jnp.float32),
        mesh=mesh,
        compiler_params=pltpu.CompilerParams(needs_layout_passes=False),
        scratch_types=[
            pltpu.VMEM((SLAB,), jnp.float32),
            pltpu.VMEM((SLAB,), jnp.float32),
            pltpu.VMEM((SLAB + 2 * L,), jnp.float32),
            pltpu.VMEM((SLAB + 2 * L,), jnp.float32),
            pltpu.VMEM((SLAB + 2 * L,), jnp.int32),
            pltpu.VMEM((A,), jnp.int32),
            pltpu.VMEM((RPW * OUTW,), jnp.float32),
            pltpu.SemaphoreType.DMA,
            pltpu.SemaphoreType.DMA,
        ],
    )


def kernel(distance_matrices_batch, num_species_batch):
    d = distance_matrices_batch.reshape(ROWS * A)
    s = num_species_batch.astype(jnp.int32).reshape(B * A)
    out = _make_sc_call()(d, s)
    return out.reshape(B, A, OUTW)
